# baseline (speedup 1.0000x reference)
import functools

import jax
import jax.numpy as jnp
from jax import lax
from jax.experimental import pallas as pl
from jax.experimental.pallas import tpu as pltpu
from jax.experimental.pallas import tpu_sc as plsc

N = 10000
E = 320000
D = 128

NC = 2
NS = 16
NW = NC * NS
L = 16

NPAD = 10240
NROWS = NPAD // 128
EPW = 10112
CH = 64
ECH = EPW // CH
NB = 4
STG = 40
ROWS_PER_TILE = NPAD // NS

_mesh = plsc.VectorSubcoreMesh(
    core_axis_name="c", subcore_axis_name="s", num_cores=NC, num_subcores=NS)


ROWS_PER_TILE_H = NROWS // NS


def _core_reduce_prologue(zeros_hbm, iota_hbm, sid, acc_v, iota_v, hist_sh):
    pltpu.sync_copy(iota_hbm, iota_v)
    pltpu.sync_copy(zeros_hbm, acc_v)
    pltpu.sync_copy(zeros_hbm.at[pl.ds(0, ROWS_PER_TILE_H)],
                    hist_sh.at[pl.ds(sid * ROWS_PER_TILE_H, ROWS_PER_TILE_H)])
    plsc.subcore_barrier()


def _core_reduce_epilogue(cid, sid, acc_v, iota_v, hist_sh, out_hbm):
    pltpu.sync_copy(acc_v, hist_sh.at[iota_v], add=True)
    plsc.subcore_barrier()

    @pl.when(sid == 0)
    def _():
        pltpu.sync_copy(hist_sh, out_hbm.at[cid])


@functools.partial(
    pl.kernel,
    out_type=jax.ShapeDtypeStruct((NC, NROWS, 128), jnp.float32),
    mesh=_mesh,
    compiler_params=pltpu.CompilerParams(needs_layout_passes=False),
    scratch_types=[
        pltpu.VMEM((EPW,), jnp.int32),
        pltpu.VMEM((NROWS, 128), jnp.float32),
        pltpu.VMEM((NROWS,), jnp.int32),
        pltpu.VMEM_SHARED((NROWS, 128), jnp.float32),
    ],
)
def _sc_hist(sidx_hbm, iota_hbm, zeros_hbm, out_hbm,
             sidx_v, acc_v, iota_v, hist_sh):
    cid = lax.axis_index("c")
    sid = lax.axis_index("s")
    wid = cid * NS + sid
    pltpu.sync_copy(sidx_hbm.at[wid], sidx_v)
    _core_reduce_prologue(zeros_hbm, iota_hbm, sid, acc_v, iota_v, hist_sh)
    ones = jnp.full((L,), 1.0, jnp.float32)

    def step(i, _):
        si = sidx_v[pl.ds(i * L, L)]
        plsc.addupdate_scatter(acc_v, [si >> 7, si & 127], ones)
        return 0

    lax.fori_loop(0, EPW // L, step, 0)
    _core_reduce_epilogue(cid, sid, acc_v, iota_v, hist_sh, out_hbm)


@functools.partial(
    pl.kernel,
    out_type=jax.ShapeDtypeStruct((NC, NROWS, 128), jnp.float32),
    mesh=_mesh,
    compiler_params=pltpu.CompilerParams(needs_layout_passes=False),
    scratch_types=[
        pltpu.VMEM((EPW,), jnp.int32),
        pltpu.VMEM((EPW,), jnp.int32),
        pltpu.VMEM((NPAD,), jnp.float32),
        pltpu.VMEM((NPAD,), jnp.float32),
        pltpu.VMEM((NROWS, 128), jnp.float32),
        pltpu.VMEM((NROWS,), jnp.int32),
        pltpu.VMEM_SHARED((NROWS, 128), jnp.float32),
    ],
)
def _sc_wg(hpart_hbm, gidx_hbm, sidx_hbm, iota_hbm, zeros_hbm, out_hbm,
           gidx_v, sidx_v, w_v, p1_v, acc_v, iota_v, hist_sh):
    cid = lax.axis_index("c")
    sid = lax.axis_index("s")
    wid = cid * NS + sid
    pltpu.sync_copy(gidx_hbm.at[wid], gidx_v)
    pltpu.sync_copy(sidx_hbm.at[wid], sidx_v)
    pltpu.sync_copy(hpart_hbm.at[0], w_v)
    pltpu.sync_copy(hpart_hbm.at[1], p1_v)
    _core_reduce_prologue(zeros_hbm, iota_hbm, sid, acc_v, iota_v, hist_sh)

    def wstep(t, _):
        d = w_v[pl.ds(t * L, L)] + p1_v[pl.ds(t * L, L)]
        w_v[pl.ds(t * L, L)] = jnp.where(d > 0, 1.0 / d, 0.0)
        return 0

    lax.fori_loop(0, NPAD // L, wstep, 0)

    def step(i, _):
        gi = gidx_v[pl.ds(i * L, L)]
        vals = plsc.load_gather(w_v, [gi])
        si = sidx_v[pl.ds(i * L, L)]
        plsc.addupdate_scatter(acc_v, [si >> 7, si & 127], vals)
        return 0

    lax.fori_loop(0, EPW // L, step, 0)
    _core_reduce_epilogue(cid, sid, acc_v, iota_v, hist_sh, out_hbm)


@functools.partial(
    pl.kernel,
    out_type=jax.ShapeDtypeStruct((NC, NROWS, 128), jnp.float32),
    mesh=_mesh,
    compiler_params=pltpu.CompilerParams(needs_layout_passes=False),
    scratch_types=[
        pltpu.VMEM((EPW,), jnp.int32),
        pltpu.VMEM((EPW,), jnp.int32),
        pltpu.VMEM((NPAD,), jnp.float32),
        pltpu.VMEM((NROWS, 128), jnp.float32),
        pltpu.VMEM((NROWS,), jnp.int32),
        pltpu.VMEM_SHARED((NROWS, 128), jnp.float32),
    ],
)
def _sc_tab_scatter(gidx_hbm, sidx_hbm, tab_hbm, iota_hbm, zeros_hbm, out_hbm,
                    gidx_v, sidx_v, tab_v, acc_v, iota_v, hist_sh):
    cid = lax.axis_index("c")
    sid = lax.axis_index("s")
    wid = cid * NS + sid
    pltpu.sync_copy(gidx_hbm.at[wid], gidx_v)
    pltpu.sync_copy(sidx_hbm.at[wid], sidx_v)
    pltpu.sync_copy(tab_hbm, tab_v)
    _core_reduce_prologue(zeros_hbm, iota_hbm, sid, acc_v, iota_v, hist_sh)

    def step(i, _):
        gi = gidx_v[pl.ds(i * L, L)]
        vals = plsc.load_gather(tab_v, [gi])
        si = sidx_v[pl.ds(i * L, L)]
        plsc.addupdate_scatter(acc_v, [si >> 7, si & 127], vals)
        return 0

    lax.fori_loop(0, EPW // L, step, 0)
    _core_reduce_epilogue(cid, sid, acc_v, iota_v, hist_sh, out_hbm)


@functools.partial(
    pl.kernel,
    out_type=jax.ShapeDtypeStruct((NC, NPAD, 128), jnp.float32),
    mesh=_mesh,
    compiler_params=pltpu.CompilerParams(needs_layout_passes=False),
    scratch_types=[
        pltpu.VMEM((STG, CH), jnp.int32),
        pltpu.VMEM((STG, CH), jnp.int32),
        pltpu.VMEM((NB, CH, 128), jnp.float32),
        pltpu.VMEM_SHARED((NPAD, 128), jnp.float32),
        pltpu.SemaphoreType.DMA,
        pltpu.SemaphoreType.DMA,
        pltpu.SemaphoreType.DMA,
        pltpu.SemaphoreType.DMA,
        pltpu.SemaphoreType.DMA,
        pltpu.SemaphoreType.DMA,
        pltpu.SemaphoreType.DMA,
        pltpu.SemaphoreType.DMA,
    ],
)
def _sc_row_scatter(u_hbm, src_hbm, dst_hbm, zeros_hbm, out_hbm,
                    sidx_v, didx_v, ring_v, acc_sh,
                    gsem0, gsem1, gsem2, gsem3, ssem0, ssem1, ssem2, ssem3):
    cid = lax.axis_index("c")
    sid = lax.axis_index("s")
    wid = cid * NS + sid
    rows = tuple(ring_v.at[b] for b in range(NB))
    gsems = (gsem0, gsem1, gsem2, gsem3)
    ssems = (ssem0, ssem1, ssem2, ssem3)
    pltpu.sync_copy(zeros_hbm, rows[0])
    base = sid * ROWS_PER_TILE
    for k in range(ROWS_PER_TILE // CH):
        pltpu.sync_copy(rows[0], acc_sh.at[pl.ds(base + k * CH, CH)])
    plsc.subcore_barrier()

    for h in range(ECH // STG + 1):
        nh = min(STG, ECH - h * STG)
        if nh <= 0:
            break
        pltpu.sync_copy(src_hbm.at[wid].at[pl.ds(h * STG, nh)],
                        sidx_v.at[pl.ds(0, nh)])
        pltpu.sync_copy(dst_hbm.at[wid].at[pl.ds(h * STG, nh)],
                        didx_v.at[pl.ds(0, nh)])
        for b in range(NB - 1):
            pltpu.async_copy(u_hbm.at[sidx_v.at[b]], rows[b], gsems[b])

        def step(j, _):
            for p in range(NB):
                @pl.when(j % NB == p)
                def _():
                    @pl.when(j + NB - 1 < nh)
                    def _():
                        q = (p + NB - 1) % NB

                        @pl.when(j >= 1)
                        def _():
                            pltpu.make_async_copy(
                                rows[q], acc_sh.at[didx_v.at[j - 1]],
                                ssems[q]).wait()

                        pltpu.async_copy(
                            u_hbm.at[sidx_v.at[j + NB - 1]], rows[q],
                            gsems[q])

                    pltpu.make_async_copy(
                        u_hbm.at[sidx_v.at[j]], rows[p], gsems[p]).wait()
                    pltpu.async_copy(
                        rows[p], acc_sh.at[didx_v.at[j]], ssems[p], add=True)

            return 0

        lax.fori_loop(0, nh, step, 0)
        for k in range(max(0, nh - NB), nh):
            pltpu.make_async_copy(
                rows[k % NB], acc_sh.at[didx_v.at[k]], ssems[k % NB]).wait()
    plsc.subcore_barrier()
    pltpu.sync_copy(acc_sh.at[pl.ds(base, ROWS_PER_TILE)],
                    out_hbm.at[cid].at[pl.ds(base, ROWS_PER_TILE)])


def _tc_hu_body(feat_ref, w1_ref, hp_ref, gp_ref,
                h_ref, u_ref, c_ref, dis2_ref):
    hp = hp_ref[...]
    dr = hp[0] + hp[1]
    w = jnp.where(dr > 0, 1.0 / dr, 0.0)
    gp = gp_ref[...]
    deg = w * (gp[0] + gp[1]) + 1.0
    dis2 = 1.0 / deg
    c = lax.rsqrt(deg) * w
    c_ref[...] = c
    dis2_ref[...] = dis2
    f = feat_ref[...]
    fn = f / jnp.sum(f, axis=1, keepdims=True)
    h = jnp.dot(fn, w1_ref[...], preferred_element_type=jnp.float32)
    h_ref[...] = h
    u_ref[...] = c * h


def _tc_z_body(ypart_ref, h_ref, c_ref, dis2_ref, b1_ref, w2_ref,
               t_ref, s2_ref):
    yp = ypart_ref[...]
    y = c_ref[...] * (yp[0] + yp[1]) + dis2_ref[...] * h_ref[...] + b1_ref[...]
    y1 = jnp.maximum(y, 0.0)
    z = jnp.dot(y1, w2_ref[...], preferred_element_type=jnp.float32)
    t_ref[...] = c_ref[...] * z
    s2_ref[...] = dis2_ref[...] * z


def _tc_out_body(zpart_ref, c_ref, s2_ref, b2_ref, out_ref):
    zs = jnp.sum(zpart_ref[...], axis=0)
    out_ref[...] = c_ref[...] * zs + s2_ref[...] + b2_ref[0, 0]


def _sds(shape):
    return jax.ShapeDtypeStruct(shape, jnp.float32)


def kernel(feat, edge_index, W1, b1, W2, b2):
    src = edge_index[0]
    dst = edge_index[1]
    pad = N + (jnp.arange(NW * EPW - E, dtype=jnp.int32) % (NPAD - N))
    src_p = jnp.concatenate([src, pad])
    dst_p = jnp.concatenate([dst, pad])
    src2 = src_p.reshape(NW, EPW)
    dst2 = dst_p.reshape(NW, EPW)
    src3 = src_p.reshape(NW, ECH, CH)
    dst3 = dst_p.reshape(NW, ECH, CH)

    zeros80 = jnp.zeros((NROWS, 128), jnp.float32)
    zeros_ch = jnp.zeros((CH, 128), jnp.float32)
    iota80 = jnp.arange(NROWS, dtype=jnp.int32)
    feat_p = jnp.concatenate(
        [feat, jnp.ones((NPAD - N, D), jnp.float32)], axis=0)

    hpart = _sc_hist(src2, iota80, zeros80)

    gpart = _sc_wg(hpart.reshape(NC, NPAD), src2, dst2, iota80, zeros80)

    BR = 512
    grid = NPAD // BR
    h, u, c_col, dis2_col = pl.pallas_call(
        _tc_hu_body,
        grid=(grid,),
        in_specs=[
            pl.BlockSpec((BR, D), lambda i: (i, 0)),
            pl.BlockSpec((D, D), lambda i: (0, 0)),
            pl.BlockSpec((NC, BR, 1), lambda i: (0, i, 0)),
            pl.BlockSpec((NC, BR, 1), lambda i: (0, i, 0)),
        ],
        out_specs=(pl.BlockSpec((BR, D), lambda i: (i, 0)),
                   pl.BlockSpec((BR, D), lambda i: (i, 0)),
                   pl.BlockSpec((BR, 1), lambda i: (i, 0)),
                   pl.BlockSpec((BR, 1), lambda i: (i, 0))),
        out_shape=(_sds((NPAD, D)), _sds((NPAD, D)),
                   _sds((NPAD, 1)), _sds((NPAD, 1))),
    )(feat_p, W1, hpart.reshape(NC, NPAD, 1), gpart.reshape(NC, NPAD, 1))

    ypart = _sc_row_scatter(u, src3, dst3, zeros_ch)

    t_col, s2_col = pl.pallas_call(
        _tc_z_body,
        grid=(grid,),
        in_specs=[
            pl.BlockSpec((NC, BR, D), lambda i: (0, i, 0)),
            pl.BlockSpec((BR, D), lambda i: (i, 0)),
            pl.BlockSpec((BR, 1), lambda i: (i, 0)),
            pl.BlockSpec((BR, 1), lambda i: (i, 0)),
            pl.BlockSpec((1, D), lambda i: (0, 0)),
            pl.BlockSpec((D, 1), lambda i: (0, 0)),
        ],
        out_specs=(pl.BlockSpec((BR, 1), lambda i: (i, 0)),
                   pl.BlockSpec((BR, 1), lambda i: (i, 0))),
        out_shape=(_sds((NPAD, 1)), _sds((NPAD, 1))),
    )(ypart, h, c_col, dis2_col, b1.reshape(1, D), W2)

    zpart = _sc_tab_scatter(src2, dst2, t_col.reshape(NPAD), iota80, zeros80)

    out80 = pl.pallas_call(
        _tc_out_body,
        out_shape=_sds((NROWS, 128)),
    )(zpart, c_col.reshape(NROWS, 128), s2_col.reshape(NROWS, 128),
      b2.reshape(1, 1))

    return out80.reshape(NPAD, 1)[:N]


# speedup vs baseline: 1.0539x; 1.0539x over previous
import functools

import jax
import jax.numpy as jnp
from jax import lax
from jax.experimental import pallas as pl
from jax.experimental.pallas import tpu as pltpu
from jax.experimental.pallas import tpu_sc as plsc

N = 10000
E = 320000
D = 128

NC = 2
NS = 16
NW = NC * NS
L = 16

NPAD = 10240
NROWS = NPAD // 128
EPW = 10112
CH = 64
ECH = EPW // CH
NB = 4
STG = 40
ROWS_PER_TILE = NPAD // NS

_mesh = plsc.VectorSubcoreMesh(
    core_axis_name="c", subcore_axis_name="s", num_cores=NC, num_subcores=NS)


@functools.partial(
    pl.kernel,
    out_type=jax.ShapeDtypeStruct((NW, NPAD), jnp.float32),
    mesh=_mesh,
    compiler_params=pltpu.CompilerParams(needs_layout_passes=False),
    scratch_types=[
        pltpu.VMEM((EPW,), jnp.int32),
        pltpu.VMEM((EPW,), jnp.int32),
        pltpu.VMEM((NPAD,), jnp.float32),
        pltpu.VMEM((NPAD,), jnp.float32),
    ],
)
def _sc_scalar_scatter(gidx_hbm, sidx_hbm, tab_hbm, zeros_hbm, out_hbm,
                       gidx_v, sidx_v, tab_v, acc_v):
    wid = lax.axis_index("c") * NS + lax.axis_index("s")
    pltpu.sync_copy(gidx_hbm.at[wid], gidx_v)
    pltpu.sync_copy(sidx_hbm.at[wid], sidx_v)
    pltpu.sync_copy(tab_hbm, tab_v)
    pltpu.sync_copy(zeros_hbm, acc_v)

    def step(i, _):
        gi = gidx_v[pl.ds(i * L, L)]
        vals = plsc.load_gather(tab_v, [gi])
        si = sidx_v[pl.ds(i * L, L)]
        plsc.addupdate_scatter(acc_v, [si], vals)
        return 0

    lax.fori_loop(0, EPW // L, step, 0)
    pltpu.sync_copy(acc_v, out_hbm.at[wid])


@functools.partial(
    pl.kernel,
    out_type=jax.ShapeDtypeStruct((NC, NPAD, 128), jnp.float32),
    mesh=_mesh,
    compiler_params=pltpu.CompilerParams(needs_layout_passes=False),
    scratch_types=[
        pltpu.VMEM((STG, CH), jnp.int32),
        pltpu.VMEM((STG, CH), jnp.int32),
        pltpu.VMEM((NB, CH, 128), jnp.float32),
        pltpu.VMEM_SHARED((NPAD, 128), jnp.float32),
        pltpu.SemaphoreType.DMA,
        pltpu.SemaphoreType.DMA,
        pltpu.SemaphoreType.DMA,
        pltpu.SemaphoreType.DMA,
        pltpu.SemaphoreType.DMA,
        pltpu.SemaphoreType.DMA,
        pltpu.SemaphoreType.DMA,
        pltpu.SemaphoreType.DMA,
    ],
)
def _sc_row_scatter(u_hbm, src_hbm, dst_hbm, zeros_hbm, out_hbm,
                    sidx_v, didx_v, ring_v, acc_sh,
                    gsem0, gsem1, gsem2, gsem3, ssem0, ssem1, ssem2, ssem3):
    cid = lax.axis_index("c")
    sid = lax.axis_index("s")
    wid = cid * NS + sid
    rows = tuple(ring_v.at[b] for b in range(NB))
    gsems = (gsem0, gsem1, gsem2, gsem3)
    ssems = (ssem0, ssem1, ssem2, ssem3)
    pltpu.sync_copy(zeros_hbm, rows[0])
    base = sid * ROWS_PER_TILE
    for k in range(ROWS_PER_TILE // CH):
        pltpu.sync_copy(rows[0], acc_sh.at[pl.ds(base + k * CH, CH)])
    plsc.subcore_barrier()

    for h in range(ECH // STG + 1):
        nh = min(STG, ECH - h * STG)
        if nh <= 0:
            break
        pltpu.sync_copy(src_hbm.at[wid].at[pl.ds(h * STG, nh)],
                        sidx_v.at[pl.ds(0, nh)])
        pltpu.sync_copy(dst_hbm.at[wid].at[pl.ds(h * STG, nh)],
                        didx_v.at[pl.ds(0, nh)])
        for b in range(NB - 1):
            pltpu.async_copy(u_hbm.at[sidx_v.at[b]], rows[b], gsems[b])

        def step(j, _):
            for p in range(NB):
                @pl.when(j % NB == p)
                def _():
                    @pl.when(j + NB - 1 < nh)
                    def _():
                        q = (p + NB - 1) % NB

                        @pl.when(j >= 1)
                        def _():
                            pltpu.make_async_copy(
                                rows[q], acc_sh.at[didx_v.at[j - 1]],
                                ssems[q]).wait()

                        pltpu.async_copy(
                            u_hbm.at[sidx_v.at[j + NB - 1]], rows[q],
                            gsems[q])

                    pltpu.make_async_copy(
                        u_hbm.at[sidx_v.at[j]], rows[p], gsems[p]).wait()
                    pltpu.async_copy(
                        rows[p], acc_sh.at[didx_v.at[j]], ssems[p], add=True)

            return 0

        lax.fori_loop(0, nh, step, 0)
        for k in range(max(0, nh - NB), nh):
            pltpu.make_async_copy(
                rows[k % NB], acc_sh.at[didx_v.at[k]], ssems[k % NB]).wait()
    plsc.subcore_barrier()
    pltpu.sync_copy(acc_sh.at[pl.ds(base, ROWS_PER_TILE)],
                    out_hbm.at[cid].at[pl.ds(base, ROWS_PER_TILE)])


def _tc_w_body(hpart_ref, dr_ref, w_ref):
    dr = jnp.sum(hpart_ref[...], axis=0)
    dr_ref[...] = dr
    w_ref[...] = jnp.where(dr > 0, 1.0 / dr, 0.0)


def _tc_c_body(gpart_ref, dr_ref, w_ref, c_ref, dis2_ref):
    g = jnp.sum(gpart_ref[...], axis=0)
    w = w_ref[...]
    deg = w * g + 1.0
    dis2 = 1.0 / deg
    c_ref[...] = lax.rsqrt(deg) * w
    dis2_ref[...] = dis2
    del dr_ref


def _tc_h_body(feat_ref, w1_ref, c_ref, h_ref, u_ref):
    f = feat_ref[...]
    fn = f / jnp.sum(f, axis=1, keepdims=True)
    h = jnp.dot(fn, w1_ref[...], preferred_element_type=jnp.float32)
    h_ref[...] = h
    u_ref[...] = c_ref[...] * h


def _tc_z_body(ypart_ref, h_ref, c_ref, dis2_ref, b1_ref, w2_ref,
               t_ref, s2_ref):
    yp = ypart_ref[...]
    y = c_ref[...] * (yp[0] + yp[1]) + dis2_ref[...] * h_ref[...] + b1_ref[...]
    y1 = jnp.maximum(y, 0.0)
    z = jnp.dot(y1, w2_ref[...], preferred_element_type=jnp.float32)
    t_ref[...] = c_ref[...] * z
    s2_ref[...] = dis2_ref[...] * z


def _tc_out_body(zpart_ref, c_ref, s2_ref, b2_ref, out_ref):
    zs = jnp.sum(zpart_ref[...], axis=0)
    out_ref[...] = c_ref[...] * zs + s2_ref[...] + b2_ref[0, 0]


def _sds(shape):
    return jax.ShapeDtypeStruct(shape, jnp.float32)


def kernel(feat, edge_index, W1, b1, W2, b2):
    src = edge_index[0]
    dst = edge_index[1]
    pad = N + (jnp.arange(NW * EPW - E, dtype=jnp.int32) % (NPAD - N))
    src_p = jnp.concatenate([src, pad])
    dst_p = jnp.concatenate([dst, pad])
    src2 = src_p.reshape(NW, EPW)
    dst2 = dst_p.reshape(NW, EPW)
    src3 = src_p.reshape(NW, ECH, CH)
    dst3 = dst_p.reshape(NW, ECH, CH)

    zeros_n = jnp.zeros((NPAD,), jnp.float32)
    zeros_ch = jnp.zeros((CH, 128), jnp.float32)
    ones_n = jnp.ones((NPAD,), jnp.float32)
    feat_p = jnp.concatenate(
        [feat, jnp.ones((NPAD - N, D), jnp.float32)], axis=0)

    hpart = _sc_scalar_scatter(src2, src2, ones_n, zeros_n)
    dr80, w80 = pl.pallas_call(
        _tc_w_body,
        out_shape=(_sds((NROWS, 128)), _sds((NROWS, 128))),
    )(hpart.reshape(NW, NROWS, 128))

    gpart = _sc_scalar_scatter(src2, dst2, w80.reshape(NPAD), zeros_n)
    c80, dis280 = pl.pallas_call(
        _tc_c_body,
        out_shape=(_sds((NROWS, 128)), _sds((NROWS, 128))),
    )(gpart.reshape(NW, NROWS, 128), dr80, w80)

    c_col = c80.reshape(NPAD, 1)
    dis2_col = dis280.reshape(NPAD, 1)

    BR = 512
    grid = NPAD // BR
    h, u = pl.pallas_call(
        _tc_h_body,
        grid=(grid,),
        in_specs=[
            pl.BlockSpec((BR, D), lambda i: (i, 0)),
            pl.BlockSpec((D, D), lambda i: (0, 0)),
            pl.BlockSpec((BR, 1), lambda i: (i, 0)),
        ],
        out_specs=(pl.BlockSpec((BR, D), lambda i: (i, 0)),
                   pl.BlockSpec((BR, D), lambda i: (i, 0))),
        out_shape=(_sds((NPAD, D)), _sds((NPAD, D))),
    )(feat_p, W1, c_col)

    ypart = _sc_row_scatter(u, src3, dst3, zeros_ch)

    t_col, s2_col = pl.pallas_call(
        _tc_z_body,
        grid=(grid,),
        in_specs=[
            pl.BlockSpec((NC, BR, D), lambda i: (0, i, 0)),
            pl.BlockSpec((BR, D), lambda i: (i, 0)),
            pl.BlockSpec((BR, 1), lambda i: (i, 0)),
            pl.BlockSpec((BR, 1), lambda i: (i, 0)),
            pl.BlockSpec((1, D), lambda i: (0, 0)),
            pl.BlockSpec((D, 1), lambda i: (0, 0)),
        ],
        out_specs=(pl.BlockSpec((BR, 1), lambda i: (i, 0)),
                   pl.BlockSpec((BR, 1), lambda i: (i, 0))),
        out_shape=(_sds((NPAD, 1)), _sds((NPAD, 1))),
    )(ypart, h, c_col, dis2_col, b1.reshape(1, D), W2)

    zpart = _sc_scalar_scatter(src2, dst2, t_col.reshape(NPAD), zeros_n)

    out80 = pl.pallas_call(
        _tc_out_body,
        out_shape=_sds((NROWS, 128)),
    )(zpart.reshape(NW, NROWS, 128), c80, s2_col.reshape(NROWS, 128),
      b2.reshape(1, 1))

    return out80.reshape(NPAD, 1)[:N]
